# baseline (speedup 1.0000x reference)
import functools

import jax
import jax.numpy as jnp
import numpy as np
from jax import lax
from jax.experimental import pallas as pl
from jax.experimental.pallas import tpu as pltpu
from jax.experimental.pallas import tpu_sc as plsc

_VOCAB = 1_000_000
_D = 64
_DP = 2 * _D
_B = 4096
_L = 50
_NC = 2
_NS = 16
_NW = _NC * _NS
_ROWS_W = _B // _NW
_POS_W = _ROWS_W * _L
_CHUNK = 128
_NCH = _POS_W // _CHUNK
_LANES = 16
_NG = _CHUNK // _LANES
_NROW = 2
_NCMP = 4
_LOOKAHEAD = 2

_DSTROW_NP = (
    (np.arange(_POS_W, dtype=np.int32) // _L)[None, :]
    + (np.arange(_NS, dtype=np.int32) * _ROWS_W)[:, None]
).reshape(_NS, _NCH, _CHUNK)

_mesh = plsc.VectorSubcoreMesh(core_axis_name="c", subcore_axis_name="s")


@functools.partial(
    pl.kernel,
    mesh=_mesh,
    compiler_params=pltpu.CompilerParams(use_tc_tiling_on_sc=False),
    out_type=(
        jax.ShapeDtypeStruct((_B, _D), jnp.float32),
        jax.ShapeDtypeStruct((_B * _L, _D), jnp.float32),
    ),
    scratch_types=(
        [pltpu.VMEM((_NCH, _CHUNK), jnp.int32)]
        + [pltpu.VMEM((_NCH, _CHUNK), jnp.int32)]
        + [pltpu.VMEM((_CHUNK,), jnp.int32) for _ in range(_NROW)]
        + [pltpu.VMEM((_CHUNK, _DP), jnp.float32) for _ in range(_NROW)]
        + [pltpu.VMEM((_CHUNK, _D), jnp.float32) for _ in range(_NCMP)]
        + [pltpu.VMEM((_ROWS_W, _D), jnp.float32)]
        + [pltpu.VMEM((_ROWS_W,), jnp.int32)]
        + [pltpu.VMEM((_ROWS_W,), jnp.float32)]
        + [pltpu.VMEM_SHARED((_NS * _ROWS_W, _D), jnp.float32)]
        + [pltpu.SemaphoreType.DMA for _ in range(_NROW + 2 * _NCMP)]
    ),
)
def _encode(table, xflat, dstrow, xlen, ret, out,
            idx_all, dsti_v, h0, h1, w0, w1, c0, c1, c2, c3,
            acc_v, leni_v, lenf_v, acc_sh,
            g0, g1, o0, o1, o2, o3, a0, a1, a2, a3):
    idxh = [h0, h1]
    wide = [w0, w1]
    cmp = [c0, c1, c2, c3]
    gsem = [g0, g1]
    osem = [o0, o1, o2, o3]
    asem = [a0, a1, a2, a3]

    c = lax.axis_index("c")
    s = lax.axis_index("s")
    wid = c * _NS + s
    base = wid * _POS_W

    zeros16 = jnp.zeros((_LANES,), jnp.float32)
    one16 = jnp.full((_LANES,), 1, jnp.int32)

    pltpu.sync_copy(xflat.at[pl.ds(wid * _NCH, _NCH)], idx_all)
    pltpu.sync_copy(dstrow.at[s], dsti_v)

    def _zero_body(r, carry):
        for k in range(_D // _LANES):
            acc_v[r, pl.ds(k * _LANES, _LANES)] = zeros16
        return carry

    lax.fori_loop(0, _ROWS_W, _zero_body, 0)
    pltpu.sync_copy(acc_v, acc_sh.at[pl.ds(s * _ROWS_W, _ROWS_W)])

    def _stage_and_gather(ci, rb):
        for g in range(_NG):
            sl = pl.ds(g * _LANES, _LANES)
            idxh[rb][sl] = lax.shift_right_logical(idx_all[ci, sl], one16)
        pltpu.async_copy(table.at[idxh[rb]], wide[rb], gsem[rb])

    for b in range(_LOOKAHEAD):
        _stage_and_gather(b, b)

    def _visit(ci, rb, cb):
        pltpu.make_async_copy(table.at[idxh[rb]], wide[rb], gsem[rb]).wait()

        @pl.when(ci >= _NCMP)
        def _drain_cmp():
            pltpu.make_async_copy(
                cmp[cb], out.at[pl.ds(base, _CHUNK)], osem[cb]).wait()
            pltpu.make_async_copy(
                cmp[cb], acc_sh.at[dsti_v.at[ci]], asem[cb]).wait()

        def _cmp_body(g, carry2):
            par = (idx_all[ci, pl.ds(g * _LANES, _LANES)] & one16) * _D
            for j in range(_LANES):
                off = par[j]
                p = g * _LANES + j
                for k in range(_D // _LANES):
                    cmp[cb][p, pl.ds(k * _LANES, _LANES)] = (
                        wide[rb][p, pl.ds(off + k * _LANES, _LANES)])
            return carry2

        lax.fori_loop(0, _NG, _cmp_body, 0)

        mn = idx_all[ci, pl.ds(0, _LANES)]
        for g in range(1, _NG):
            mn = jnp.minimum(mn, idx_all[ci, pl.ds(g * _LANES, _LANES)])
        anyz = (mn[0] == 0)
        for j in range(1, _LANES):
            anyz = anyz | (mn[j] == 0)

        @pl.when(anyz)
        def _fixup():
            def _grp_body(g, carry2):
                vg = idx_all[ci, pl.ds(g * _LANES, _LANES)]
                for j in range(_LANES):
                    @pl.when(vg[j] == 0)
                    def _zero_row(j=j):
                        p = g * _LANES + j
                        for k in range(_D // _LANES):
                            cmp[cb][p, pl.ds(k * _LANES, _LANES)] = zeros16
                return carry2

            lax.fori_loop(0, _NG, _grp_body, 0)

        pltpu.async_copy(cmp[cb], acc_sh.at[dsti_v.at[ci]], asem[cb],
                         add=True)
        pltpu.async_copy(cmp[cb], out.at[pl.ds(base + ci * _CHUNK, _CHUNK)],
                         osem[cb])

        nxt = ci + _LOOKAHEAD

        @pl.when(nxt < _NCH)
        def _prefetch():
            _stage_and_gather(nxt, (rb + _LOOKAHEAD) % _NROW)

    def _round_body(i, carry):
        for b in range(_NROW * _NCMP):
            ci = i * (_NROW * _NCMP) + b

            @pl.when(ci < _NCH)
            def _guarded():
                _visit(ci, b % _NROW, b % _NCMP)
        return carry

    nper = _NROW * _NCMP
    lax.fori_loop(0, (_NCH + nper - 1) // nper, _round_body, 0)

    for b in range(_NCMP):
        pltpu.make_async_copy(cmp[b],
                              out.at[pl.ds(base, _CHUNK)], osem[b]).wait()
        pltpu.make_async_copy(cmp[b], acc_sh.at[dsti_v.at[0]],
                              asem[b]).wait()

    pltpu.sync_copy(acc_sh.at[pl.ds(s * _ROWS_W, _ROWS_W)], acc_v)
    pltpu.sync_copy(xlen.at[pl.ds(wid * _ROWS_W, _ROWS_W)], leni_v)
    for g in range(_ROWS_W // _LANES):
        sl = pl.ds(g * _LANES, _LANES)
        lenf_v[sl] = leni_v[sl].astype(jnp.float32)

    def _div_body(g, carry):
        lvec = lenf_v[pl.ds(g * _LANES, _LANES)]
        for j in range(_LANES):
            bc = jnp.full((_LANES,), lvec[j], jnp.float32)
            r = g * _LANES + j
            for k in range(_D // _LANES):
                sl = pl.ds(k * _LANES, _LANES)
                acc_v[r, sl] = acc_v[r, sl] / bc
        return carry

    lax.fori_loop(0, _ROWS_W // _LANES, _div_body, 0)
    pltpu.sync_copy(acc_v, ret.at[pl.ds(wid * _ROWS_W, _ROWS_W)])


def kernel(emb_table, x, x_len):
    table2 = emb_table.reshape(_VOCAB // 2, _DP)
    xf = x.reshape(-1, _CHUNK).astype(jnp.int32)
    xl = x_len.astype(jnp.int32)
    dstrow = jnp.asarray(_DSTROW_NP)
    ret, out = _encode(table2, xf, dstrow, xl)
    return ret, out.reshape(_B, _L, _D)


# speedup vs baseline: 1.0880x; 1.0880x over previous
import functools

import jax
import jax.numpy as jnp
import numpy as np
from jax import lax
from jax.experimental import pallas as pl
from jax.experimental.pallas import tpu as pltpu
from jax.experimental.pallas import tpu_sc as plsc

_VOCAB = 1_000_000
_D = 64
_B = 4096
_L = 50
_NC = 2
_NS = 16
_NW = _NC * _NS
_ROWS_W = _B // _NW
_POS_W = _ROWS_W * _L
_CHUNK = 128
_NCH = _POS_W // _CHUNK
_LANES = 16
_NG = _CHUNK // _LANES
_NBUF = 8
_LOOKAHEAD = 4

_DSTROW_NP = (
    (np.arange(_POS_W, dtype=np.int32) // _L)[None, :]
    + (np.arange(_NS, dtype=np.int32) * _ROWS_W)[:, None]
).reshape(_NS, _NCH, _CHUNK)

_mesh = plsc.VectorSubcoreMesh(core_axis_name="c", subcore_axis_name="s")


@functools.partial(
    pl.kernel,
    mesh=_mesh,
    compiler_params=pltpu.CompilerParams(use_tc_tiling_on_sc=False),
    out_type=(
        jax.ShapeDtypeStruct((_B, _D), jnp.float32),
        jax.ShapeDtypeStruct((_B * _L, _D), jnp.float32),
    ),
    scratch_types=(
        [pltpu.VMEM((_NCH, _CHUNK), jnp.int32)]
        + [pltpu.VMEM((_NCH, _CHUNK), jnp.int32)]
        + [pltpu.VMEM((_CHUNK, _D), jnp.float32) for _ in range(_NBUF)]
        + [pltpu.VMEM((_ROWS_W, _D), jnp.float32)]
        + [pltpu.VMEM((_ROWS_W,), jnp.int32)]
        + [pltpu.VMEM((_ROWS_W,), jnp.float32)]
        + [pltpu.VMEM_SHARED((_NS * _ROWS_W, _D), jnp.float32)]
        + [pltpu.SemaphoreType.DMA for _ in range(3 * _NBUF)]
    ),
)
def _encode(table, xflat, dstrow, xlen, ret, out,
            idx_all, dsti_v, r0, r1, r2, r3, r4, r5, r6, r7,
            acc_v, leni_v, lenf_v, acc_sh,
            g0, g1, g2, g3, g4, g5, g6, g7,
            o0, o1, o2, o3, o4, o5, o6, o7,
            a0, a1, a2, a3, a4, a5, a6, a7):
    rows = [r0, r1, r2, r3, r4, r5, r6, r7]
    gsem = [g0, g1, g2, g3, g4, g5, g6, g7]
    osem = [o0, o1, o2, o3, o4, o5, o6, o7]
    asem = [a0, a1, a2, a3, a4, a5, a6, a7]

    c = lax.axis_index("c")
    s = lax.axis_index("s")
    wid = c * _NS + s
    base = wid * _POS_W

    zeros16 = jnp.zeros((_LANES,), jnp.float32)

    pltpu.sync_copy(xflat.at[pl.ds(wid * _NCH, _NCH)], idx_all)
    pltpu.sync_copy(dstrow.at[s], dsti_v)

    def _zero_body(r, carry):
        for k in range(_D // _LANES):
            acc_v[r, pl.ds(k * _LANES, _LANES)] = zeros16
        return carry

    lax.fori_loop(0, _ROWS_W, _zero_body, 0)
    pltpu.sync_copy(acc_v, acc_sh.at[pl.ds(s * _ROWS_W, _ROWS_W)])

    for b in range(_LOOKAHEAD):
        pltpu.async_copy(table.at[idx_all.at[b]], rows[b], gsem[b])

    def _visit(ci, b):
        pltpu.make_async_copy(table.at[idx_all.at[ci]], rows[b],
                              gsem[b]).wait()

        mn = idx_all[ci, pl.ds(0, _LANES)]
        for g in range(1, _NG):
            mn = jnp.minimum(mn, idx_all[ci, pl.ds(g * _LANES, _LANES)])
        anyz = (mn[0] == 0)
        for j in range(1, _LANES):
            anyz = anyz | (mn[j] == 0)

        @pl.when(anyz)
        def _fixup():
            def _grp_body(g, carry2):
                vg = idx_all[ci, pl.ds(g * _LANES, _LANES)]
                for j in range(_LANES):
                    @pl.when(vg[j] == 0)
                    def _zero_row(j=j):
                        p = g * _LANES + j
                        for k in range(_D // _LANES):
                            rows[b][p, pl.ds(k * _LANES, _LANES)] = zeros16
                return carry2

            lax.fori_loop(0, _NG, _grp_body, 0)

        pltpu.async_copy(rows[b], acc_sh.at[dsti_v.at[ci]], asem[b],
                         add=True)
        pltpu.async_copy(rows[b], out.at[pl.ds(base + ci * _CHUNK, _CHUNK)],
                         osem[b])

        b2 = (b + _LOOKAHEAD) % _NBUF
        nxt = ci + _LOOKAHEAD

        @pl.when(nxt < _NCH)
        def _prefetch():
            @pl.when(ci >= _NBUF - _LOOKAHEAD)
            def _drain_prev():
                pltpu.make_async_copy(
                    rows[b2],
                    out.at[pl.ds(base + ci * _CHUNK, _CHUNK)],
                    osem[b2]).wait()
                pltpu.make_async_copy(
                    rows[b2], acc_sh.at[dsti_v.at[ci]], asem[b2]).wait()

            pltpu.async_copy(table.at[idx_all.at[nxt]], rows[b2], gsem[b2])

    def _round_body(i, carry):
        for b in range(_NBUF):
            ci = i * _NBUF + b

            @pl.when(ci < _NCH)
            def _guarded():
                _visit(ci, b)
        return carry

    lax.fori_loop(0, (_NCH + _NBUF - 1) // _NBUF, _round_body, 0)

    for b in range(_NBUF):
        pltpu.make_async_copy(rows[b],
                              out.at[pl.ds(base, _CHUNK)], osem[b]).wait()
        pltpu.make_async_copy(rows[b], acc_sh.at[dsti_v.at[0]],
                              asem[b]).wait()

    pltpu.sync_copy(acc_sh.at[pl.ds(s * _ROWS_W, _ROWS_W)], acc_v)
    pltpu.sync_copy(xlen.at[pl.ds(wid * _ROWS_W, _ROWS_W)], leni_v)
    for g in range(_ROWS_W // _LANES):
        sl = pl.ds(g * _LANES, _LANES)
        lenf_v[sl] = leni_v[sl].astype(jnp.float32)

    def _div_body(g, carry):
        lvec = lenf_v[pl.ds(g * _LANES, _LANES)]
        for j in range(_LANES):
            bc = jnp.full((_LANES,), lvec[j], jnp.float32)
            r = g * _LANES + j
            for k in range(_D // _LANES):
                sl = pl.ds(k * _LANES, _LANES)
                acc_v[r, sl] = acc_v[r, sl] / bc
        return carry

    lax.fori_loop(0, _ROWS_W // _LANES, _div_body, 0)
    pltpu.sync_copy(acc_v, ret.at[pl.ds(wid * _ROWS_W, _ROWS_W)])


def kernel(emb_table, x, x_len):
    xf = x.reshape(-1, _CHUNK).astype(jnp.int32)
    xl = x_len.astype(jnp.int32)
    dstrow = jnp.asarray(_DSTROW_NP)
    ret, out = _encode(emb_table, xf, dstrow, xl)
    return ret, out.reshape(_B, _L, _D)


# speedup vs baseline: 1.0888x; 1.0008x over previous
import functools

import jax
import jax.numpy as jnp
import numpy as np
from jax import lax
from jax.experimental import pallas as pl
from jax.experimental.pallas import tpu as pltpu
from jax.experimental.pallas import tpu_sc as plsc

_VOCAB = 1_000_000
_D = 64
_B = 4096
_L = 50
_NC = 2
_NS = 16
_NW = _NC * _NS
_ROWS_W = _B // _NW
_POS_W = _ROWS_W * _L
_CHUNK = 128
_NCH = _POS_W // _CHUNK
_LANES = 16
_NG = _CHUNK // _LANES
_NBUF = 8
_LOOKAHEAD = 6

_DSTROW_NP = (
    (np.arange(_POS_W, dtype=np.int32) // _L)[None, :]
    + (np.arange(_NS, dtype=np.int32) * _ROWS_W)[:, None]
).reshape(_NS, _NCH, _CHUNK)

_mesh = plsc.VectorSubcoreMesh(core_axis_name="c", subcore_axis_name="s")


@functools.partial(
    pl.kernel,
    mesh=_mesh,
    compiler_params=pltpu.CompilerParams(use_tc_tiling_on_sc=False),
    out_type=(
        jax.ShapeDtypeStruct((_B, _D), jnp.float32),
        jax.ShapeDtypeStruct((_B * _L, _D), jnp.float32),
    ),
    scratch_types=(
        [pltpu.VMEM((_NCH, _CHUNK), jnp.int32)]
        + [pltpu.VMEM((_NCH, _CHUNK), jnp.int32)]
        + [pltpu.VMEM((_CHUNK, _D), jnp.float32) for _ in range(_NBUF)]
        + [pltpu.VMEM((_ROWS_W, _D), jnp.float32)]
        + [pltpu.VMEM((_ROWS_W,), jnp.int32)]
        + [pltpu.VMEM((_ROWS_W,), jnp.float32)]
        + [pltpu.VMEM_SHARED((_NS * _ROWS_W, _D), jnp.float32)]
        + [pltpu.SemaphoreType.DMA for _ in range(3 * _NBUF)]
    ),
)
def _encode(table, xflat, dstrow, xlen, ret, out,
            idx_all, dsti_v, r0, r1, r2, r3, r4, r5, r6, r7,
            acc_v, leni_v, lenf_v, acc_sh,
            g0, g1, g2, g3, g4, g5, g6, g7,
            o0, o1, o2, o3, o4, o5, o6, o7,
            a0, a1, a2, a3, a4, a5, a6, a7):
    rows = [r0, r1, r2, r3, r4, r5, r6, r7]
    gsem = [g0, g1, g2, g3, g4, g5, g6, g7]
    osem = [o0, o1, o2, o3, o4, o5, o6, o7]
    asem = [a0, a1, a2, a3, a4, a5, a6, a7]

    c = lax.axis_index("c")
    s = lax.axis_index("s")
    wid = c * _NS + s
    base = wid * _POS_W

    zeros16 = jnp.zeros((_LANES,), jnp.float32)

    pltpu.sync_copy(xflat.at[pl.ds(wid * _NCH, _NCH)], idx_all)
    pltpu.sync_copy(dstrow.at[s], dsti_v)

    def _zero_body(r, carry):
        for k in range(_D // _LANES):
            acc_v[r, pl.ds(k * _LANES, _LANES)] = zeros16
        return carry

    lax.fori_loop(0, _ROWS_W, _zero_body, 0)
    pltpu.sync_copy(acc_v, acc_sh.at[pl.ds(s * _ROWS_W, _ROWS_W)])

    for b in range(_LOOKAHEAD):
        pltpu.async_copy(table.at[idx_all.at[b]], rows[b], gsem[b])

    def _visit(ci, b):
        pltpu.make_async_copy(table.at[idx_all.at[ci]], rows[b],
                              gsem[b]).wait()

        mn = idx_all[ci, pl.ds(0, _LANES)]
        for g in range(1, _NG):
            mn = jnp.minimum(mn, idx_all[ci, pl.ds(g * _LANES, _LANES)])
        anyz = (mn[0] == 0)
        for j in range(1, _LANES):
            anyz = anyz | (mn[j] == 0)

        @pl.when(anyz)
        def _fixup():
            def _grp_body(g, carry2):
                vg = idx_all[ci, pl.ds(g * _LANES, _LANES)]
                for j in range(_LANES):
                    @pl.when(vg[j] == 0)
                    def _zero_row(j=j):
                        p = g * _LANES + j
                        for k in range(_D // _LANES):
                            rows[b][p, pl.ds(k * _LANES, _LANES)] = zeros16
                return carry2

            lax.fori_loop(0, _NG, _grp_body, 0)

        pltpu.async_copy(rows[b], acc_sh.at[dsti_v.at[ci]], asem[b],
                         add=True)
        pltpu.async_copy(rows[b], out.at[pl.ds(base + ci * _CHUNK, _CHUNK)],
                         osem[b])

        b2 = (b + _LOOKAHEAD) % _NBUF
        nxt = ci + _LOOKAHEAD

        @pl.when(nxt < _NCH)
        def _prefetch():
            @pl.when(ci >= _NBUF - _LOOKAHEAD)
            def _drain_prev():
                pltpu.make_async_copy(
                    rows[b2],
                    out.at[pl.ds(base + ci * _CHUNK, _CHUNK)],
                    osem[b2]).wait()
                pltpu.make_async_copy(
                    rows[b2], acc_sh.at[dsti_v.at[ci]], asem[b2]).wait()

            pltpu.async_copy(table.at[idx_all.at[nxt]], rows[b2], gsem[b2])

    def _round_body(i, carry):
        for b in range(_NBUF):
            ci = i * _NBUF + b

            @pl.when(ci < _NCH)
            def _guarded():
                _visit(ci, b)
        return carry

    lax.fori_loop(0, (_NCH + _NBUF - 1) // _NBUF, _round_body, 0)

    for b in range(_NBUF):
        pltpu.make_async_copy(rows[b],
                              out.at[pl.ds(base, _CHUNK)], osem[b]).wait()
        pltpu.make_async_copy(rows[b], acc_sh.at[dsti_v.at[0]],
                              asem[b]).wait()

    pltpu.sync_copy(acc_sh.at[pl.ds(s * _ROWS_W, _ROWS_W)], acc_v)
    pltpu.sync_copy(xlen.at[pl.ds(wid * _ROWS_W, _ROWS_W)], leni_v)
    for g in range(_ROWS_W // _LANES):
        sl = pl.ds(g * _LANES, _LANES)
        lenf_v[sl] = leni_v[sl].astype(jnp.float32)

    def _div_body(g, carry):
        lvec = lenf_v[pl.ds(g * _LANES, _LANES)]
        for j in range(_LANES):
            bc = jnp.full((_LANES,), lvec[j], jnp.float32)
            r = g * _LANES + j
            for k in range(_D // _LANES):
                sl = pl.ds(k * _LANES, _LANES)
                acc_v[r, sl] = acc_v[r, sl] / bc
        return carry

    lax.fori_loop(0, _ROWS_W // _LANES, _div_body, 0)
    pltpu.sync_copy(acc_v, ret.at[pl.ds(wid * _ROWS_W, _ROWS_W)])


def kernel(emb_table, x, x_len):
    xf = x.reshape(-1, _CHUNK).astype(jnp.int32)
    xl = x_len.astype(jnp.int32)
    dstrow = jnp.asarray(_DSTROW_NP)
    ret, out = _encode(emb_table, xf, dstrow, xl)
    return ret, out.reshape(_B, _L, _D)
